# baseline (speedup 1.0000x reference)
import jax
import jax.numpy as jnp
from jax.experimental import pallas as pl
from jax.experimental.pallas import tpu as pltpu

VOCAB_SIZE = 1000


def _onehot_kernel(idxv_ref, bank_ref, out_ref):
    bb, s_per, n_words, vocab = out_ref.shape
    n = bb * s_per
    k = bank_ref.shape[1]
    idxv = idxv_ref[0]
    kio = jax.lax.broadcasted_iota(jnp.int32, (k, 1), 0)
    e = (kio == idxv).astype(jnp.float32)
    toks = jax.lax.dot(bank_ref[...], e,
                       precision=jax.lax.Precision.HIGHEST,
                       preferred_element_type=jnp.float32)
    toks = toks.astype(jnp.int32)
    col = jax.lax.broadcasted_iota(jnp.int32, (n_words, vocab), 1)
    for j in range(n):
        tok_col = toks[:, j:j + 1]
        out_ref[j // s_per, j % s_per] = (tok_col == col).astype(jnp.float32)


def kernel(features, reports, sentence_bank, weights):
    B, n_sentences, n_words = reports.shape
    k_first, bank_w = sentence_bank.shape

    key = jax.random.key(42)
    idx = jax.random.categorical(key, jnp.log(weights), shape=(B, n_sentences))
    idx = idx.astype(jnp.int32)

    if bank_w < n_words:
        sentence_bank = jnp.pad(sentence_bank, ((0, 0), (0, n_words - bank_w)))
    bank_t = sentence_bank[:, :n_words].T.astype(jnp.float32)

    bb = 8
    grid = (B // bb,)
    idxv = idx.reshape(B // bb, 1, bb * n_sentences)
    out = pl.pallas_call(
        _onehot_kernel,
        grid=grid,
        in_specs=[
            pl.BlockSpec((1, 1, bb * n_sentences), lambda i: (i, 0, 0)),
            pl.BlockSpec((n_words, k_first), lambda i: (0, 0)),
        ],
        out_specs=pl.BlockSpec(
            (bb, n_sentences, n_words, VOCAB_SIZE), lambda i: (i, 0, 0, 0)
        ),
        out_shape=jax.ShapeDtypeStruct(
            (B, n_sentences, n_words, VOCAB_SIZE), jnp.float32
        ),
    )(idxv, bank_t)

    stops = jnp.zeros((B, n_sentences), dtype=jnp.float32)
    return (out, stops)
